# baseline (speedup 1.0000x reference)
import functools

import jax
import jax.numpy as jnp
from jax import lax
from jax.experimental import pallas as pl
from jax.experimental.pallas import tpu as pltpu
from jax.experimental.pallas import tpu_sc as plsc

N_NODES = 10000
D_IN = 128
N_CLS = 3

NC = 2
NS = 16
NW = NC * NS

N_PAD = 10240
ROWS_W = N_PAD // NW
E_PAD = 327680
CHUNKS_W = 10
CHUNK = 1024
EDGES_W = CHUNKS_W * CHUNK
ROWS_OUT = N_PAD // NS

_sc_mesh = plsc.VectorSubcoreMesh(core_axis_name="c", subcore_axis_name="s")


ROWS_C0 = 160
ROWS_C1 = 480


@functools.partial(
    pl.kernel,
    out_type=jax.ShapeDtypeStruct((N_PAD, D_IN), jnp.float32),
    mesh=_sc_mesh,
    scratch_types=[
        pltpu.VMEM((max(ROWS_C0, ROWS_C1),), jnp.int32),
        pltpu.VMEM((max(ROWS_C0, ROWS_C1), D_IN), jnp.float32),
        pltpu.SemaphoreType.DMA,
    ],
    compiler_params=pltpu.CompilerParams(use_tc_tiling_on_sc=False),
)
def _gather_rows(x_hbm, nid_hbm, xs_hbm, idx_v, rows_v, sem):
    cid = lax.axis_index("c")
    sid = lax.axis_index("s")

    def work(base, n, q):
        pltpu.sync_copy(nid_hbm.at[pl.ds(base, n)], idx_v.at[pl.ds(0, n)])
        handles = [
            pltpu.async_copy(x_hbm.at[idx_v.at[pl.ds(i * q, q)]],
                             rows_v.at[pl.ds(i * q, q)], sem)
            for i in range(n // q)
        ]
        for h in handles:
            h.wait()
        pltpu.sync_copy(rows_v.at[pl.ds(0, n)], xs_hbm.at[pl.ds(base, n)])

    @pl.when(cid == 0)
    def _():
        work(sid * ROWS_C0, ROWS_C0, ROWS_C0 // 2)

    @pl.when(cid == 1)
    def _():
        work(NS * ROWS_C0 + sid * ROWS_C1, ROWS_C1, ROWS_C1 // 4)


def _project_body(xs_ref, w_sage_ref, w_cls_ref, z4_ref):
    w_fused = jnp.dot(w_sage_ref[...], w_cls_ref[...],
                      preferred_element_type=jnp.float32)
    z = jnp.dot(xs_ref[...], w_fused, preferred_element_type=jnp.float32)
    z4_ref[:, 0:N_CLS] = z
    z4_ref[:, N_CLS:4] = jnp.ones((z.shape[0], 1), jnp.float32)
    z4_ref[:, 4:8] = jnp.zeros((z.shape[0], 4), jnp.float32)


def _project(xs, w_sage, w_cls):
    return pl.pallas_call(
        _project_body,
        out_shape=jax.ShapeDtypeStruct((N_PAD, 8), jnp.float32),
    )(xs, w_sage, w_cls)


@functools.partial(
    pl.kernel,
    out_type=jax.ShapeDtypeStruct((NC, N_PAD, 8), jnp.float32),
    mesh=_sc_mesh,
    scratch_types=[
        pltpu.VMEM((CHUNKS_W, CHUNK), jnp.int32),
        pltpu.VMEM((CHUNKS_W, CHUNK), jnp.int32),
        pltpu.VMEM((EDGES_W, 8), jnp.float32),
        pltpu.VMEM_SHARED((N_PAD, 8), jnp.float32),
        pltpu.SemaphoreType.DMA((CHUNKS_W,)),
        pltpu.SemaphoreType.DMA,
    ],
    compiler_params=pltpu.CompilerParams(use_tc_tiling_on_sc=False),
)
def _edge_aggregate(src_hbm, dst_hbm, zeros_hbm, z4_hbm, acc_hbm,
                    src_v, dst_v, upd_v, acc_sh, gsems, ssem):
    cid = lax.axis_index("c")
    sid = lax.axis_index("s")
    wid = cid * NS + sid

    pltpu.sync_copy(zeros_hbm.at[pl.ds(sid * ROWS_OUT, ROWS_OUT)],
                    acc_sh.at[pl.ds(sid * ROWS_OUT, ROWS_OUT)])
    pltpu.sync_copy(src_hbm.at[wid], src_v)
    pltpu.sync_copy(dst_hbm.at[wid], dst_v)
    plsc.subcore_barrier()

    for j in range(CHUNKS_W):
        pltpu.async_copy(z4_hbm.at[src_v.at[j]],
                         upd_v.at[pl.ds(j * CHUNK, CHUNK)], gsems.at[j])
    for j in range(CHUNKS_W):
        pltpu.make_async_copy(z4_hbm.at[src_v.at[j]],
                              upd_v.at[pl.ds(j * CHUNK, CHUNK)],
                              gsems.at[j]).wait()
        pltpu.async_copy(upd_v.at[pl.ds(j * CHUNK, CHUNK)],
                         acc_sh.at[dst_v.at[j]], ssem, add=True)
    for j in range(CHUNKS_W):
        pltpu.make_async_copy(upd_v.at[pl.ds(j * CHUNK, CHUNK)],
                              acc_sh.at[dst_v.at[j]], ssem).wait()

    plsc.subcore_barrier()
    pltpu.sync_copy(acc_sh.at[pl.ds(sid * ROWS_OUT, ROWS_OUT)],
                    acc_hbm.at[cid].at[pl.ds(sid * ROWS_OUT, ROWS_OUT)])


def _finalize_body(acc_ref, w_cls_ref, b_sage_ref, b_cls_ref, out_ref):
    acc = acc_ref[0] + acc_ref[1]
    cnt = jnp.clip(acc[:N_NODES, 3:4], 1.0, None)
    b_eff = jnp.dot(b_sage_ref[...].reshape(1, -1), w_cls_ref[...],
                    preferred_element_type=jnp.float32) + b_cls_ref[...].reshape(1, -1)
    s = acc[:N_NODES, 0:N_CLS] / cnt + b_eff
    m = jnp.max(s, axis=1, keepdims=True)
    lse = jnp.log(jnp.sum(jnp.exp(s - m), axis=1, keepdims=True)) + m
    out_ref[...] = s - lse


def _finalize(acc, w_cls, b_sage, b_cls):
    return pl.pallas_call(
        _finalize_body,
        out_shape=jax.ShapeDtypeStruct((N_NODES, N_CLS), jnp.float32),
    )(acc, w_cls, b_sage, b_cls)


def kernel(x, n_id, edge_index, W_sage, b_sage, W_cls, b_cls):
    E = edge_index.shape[1]
    nid_pad = jnp.concatenate(
        [n_id, jnp.zeros((N_PAD - N_NODES,), jnp.int32)])
    src_pad = jnp.concatenate(
        [edge_index[0], jnp.zeros((E_PAD - E,), jnp.int32)]).reshape(NW, CHUNKS_W, CHUNK)
    dst_pad = jnp.concatenate(
        [edge_index[1], jnp.full((E_PAD - E,), N_PAD - 1, jnp.int32)]).reshape(NW, CHUNKS_W, CHUNK)
    zeros_init = jnp.zeros((N_PAD, 8), jnp.float32)

    xs = _gather_rows(x, nid_pad)
    z4 = _project(xs, W_sage, W_cls)
    acc = _edge_aggregate(src_pad, dst_pad, zeros_init, z4)
    return _finalize(acc, W_cls, b_sage, b_cls)


# speedup vs baseline: 1.5034x; 1.5034x over previous
import functools

import jax
import jax.numpy as jnp
from jax import lax
from jax.experimental import pallas as pl
from jax.experimental.pallas import tpu as pltpu
from jax.experimental.pallas import tpu_sc as plsc

N_NODES = 10000
D_IN = 128
N_CLS = 3
E_TOTAL = 320000

NC = 2
NS = 16
N_PAD = 10240

ROWS_C0 = 480
ROWS_C1 = 160
RCHUNK = 80

EDGES_C0 = 14400
EDGES_C1 = 5600
ECHUNK = 800
NSLOT = 4
ROWS_OUT = N_PAD // NS

_sc_mesh = plsc.VectorSubcoreMesh(core_axis_name="c", subcore_axis_name="s")
_sc_params = pltpu.CompilerParams(use_tc_tiling_on_sc=False)


@functools.partial(
    pl.kernel,
    out_type=jax.ShapeDtypeStruct((N_PAD, D_IN), jnp.float32),
    mesh=_sc_mesh,
    scratch_types=[
        pltpu.VMEM((ROWS_C0,), jnp.int32),
        pltpu.VMEM((ROWS_C0, D_IN), jnp.float32),
        pltpu.SemaphoreType.DMA,
        pltpu.SemaphoreType.DMA,
    ],
    compiler_params=_sc_params,
)
def _gather_rows(x_hbm, nid_hbm, xs_hbm, idx_v, rows_v, gsem, wsem):
    cid = lax.axis_index("c")
    sid = lax.axis_index("s")

    def work(base, n):
        nb = n // RCHUNK
        pltpu.sync_copy(nid_hbm.at[pl.ds(base, n)], idx_v.at[pl.ds(0, n)])
        for i in range(nb):
            pltpu.async_copy(x_hbm.at[idx_v.at[pl.ds(i * RCHUNK, RCHUNK)]],
                             rows_v.at[pl.ds(i * RCHUNK, RCHUNK)], gsem)
        for i in range(nb):
            pltpu.make_async_copy(
                x_hbm.at[idx_v.at[pl.ds(i * RCHUNK, RCHUNK)]],
                rows_v.at[pl.ds(i * RCHUNK, RCHUNK)], gsem).wait()
            pltpu.async_copy(rows_v.at[pl.ds(i * RCHUNK, RCHUNK)],
                             xs_hbm.at[pl.ds(base + i * RCHUNK, RCHUNK)], wsem)
        for i in range(nb):
            pltpu.make_async_copy(
                rows_v.at[pl.ds(i * RCHUNK, RCHUNK)],
                xs_hbm.at[pl.ds(base + i * RCHUNK, RCHUNK)], wsem).wait()

    @pl.when(cid == 0)
    def _():
        work(sid * ROWS_C0, ROWS_C0)

    @pl.when(cid == 1)
    def _():
        work(NS * ROWS_C0 + sid * ROWS_C1, ROWS_C1)


def _project_body(xs_ref, w_sage_ref, w_cls_ref, z8_ref):
    w_fused = jnp.dot(w_sage_ref[...], w_cls_ref[...],
                      preferred_element_type=jnp.float32)
    z = jnp.dot(xs_ref[...], w_fused, preferred_element_type=jnp.float32)
    z8_ref[:, 0:N_CLS] = z
    z8_ref[:, N_CLS:4] = jnp.ones((z.shape[0], 1), jnp.float32)
    z8_ref[:, 4:8] = jnp.zeros((z.shape[0], 4), jnp.float32)


def _project(xs, w_sage, w_cls):
    return pl.pallas_call(
        _project_body,
        out_shape=jax.ShapeDtypeStruct((N_PAD, 8), jnp.float32),
    )(xs, w_sage, w_cls)


@functools.partial(
    pl.kernel,
    out_type=jax.ShapeDtypeStruct((NC, N_PAD, 8), jnp.float32),
    mesh=_sc_mesh,
    scratch_types=[
        pltpu.VMEM((EDGES_C0,), jnp.int32),
        pltpu.VMEM((EDGES_C0 // ECHUNK, ECHUNK), jnp.int32),
        pltpu.VMEM((NSLOT, ECHUNK, 8), jnp.float32),
        pltpu.VMEM_SHARED((N_PAD, 8), jnp.float32),
        pltpu.SemaphoreType.DMA,
        pltpu.SemaphoreType.DMA((NSLOT,)),
        pltpu.SemaphoreType.DMA((NSLOT,)),
    ],
    compiler_params=_sc_params,
)
def _edge_aggregate(ei_hbm, zeros_hbm, z8_hbm, acc_hbm,
                    src_v, dst_v, upd_v, acc_sh, isem, gsems, ssems):
    cid = lax.axis_index("c")
    sid = lax.axis_index("s")

    pltpu.sync_copy(zeros_hbm.at[pl.ds(sid * ROWS_OUT, ROWS_OUT)],
                    acc_sh.at[pl.ds(sid * ROWS_OUT, ROWS_OUT)])

    def work(e_base, n_edges):
        nb = n_edges // ECHUNK
        pltpu.async_copy(ei_hbm.at[0].at[pl.ds(e_base, n_edges)],
                         src_v.at[pl.ds(0, n_edges)], isem)
        for j in range(nb):
            pltpu.async_copy(ei_hbm.at[1].at[pl.ds(e_base + j * ECHUNK, ECHUNK)],
                             dst_v.at[j], isem)
        pltpu.make_async_copy(ei_hbm.at[0].at[pl.ds(e_base, n_edges)],
                              src_v.at[pl.ds(0, n_edges)], isem).wait()
        for j in range(nb):
            pltpu.make_async_copy(
                ei_hbm.at[1].at[pl.ds(e_base + j * ECHUNK, ECHUNK)],
                dst_v.at[j], isem).wait()
        plsc.subcore_barrier()

        def gather_desc(j, slot):
            return pltpu.make_async_copy(
                z8_hbm.at[src_v.at[pl.ds(j * ECHUNK, ECHUNK)]],
                upd_v.at[slot], gsems.at[slot])

        def scatter_desc(j, slot):
            return pltpu.make_async_copy(
                upd_v.at[slot], acc_sh.at[dst_v.at[j]], ssems.at[slot])

        for i in range(nb + 2):
            if i < nb:
                slot = i % NSLOT
                if i >= NSLOT:
                    scatter_desc(i - NSLOT, slot).wait()
                gather_desc(i, slot).start()
            k = i - 2
            if 0 <= k < nb:
                slot = k % NSLOT
                gather_desc(k, slot).wait()
                pltpu.async_copy(upd_v.at[slot], acc_sh.at[dst_v.at[k]],
                                 ssems.at[slot], add=True)
        for k in range(max(0, nb - NSLOT), nb):
            scatter_desc(k, k % NSLOT).wait()

    @pl.when(cid == 0)
    def _():
        work(sid * EDGES_C0, EDGES_C0)

    @pl.when(cid == 1)
    def _():
        work(NS * EDGES_C0 + sid * EDGES_C1, EDGES_C1)

    plsc.subcore_barrier()
    pltpu.sync_copy(acc_sh.at[pl.ds(sid * ROWS_OUT, ROWS_OUT)],
                    acc_hbm.at[cid].at[pl.ds(sid * ROWS_OUT, ROWS_OUT)])


def _roll(x, k):
    k = k % x.shape[1]
    return jnp.concatenate([x[:, -k:], x[:, :-k]], axis=1)


def _finalize_body(acc_ref, w_cls_ref, b_sage_ref, b_cls_ref, out_ref):
    a = acc_ref[0] + acc_ref[1]
    lane = lax.broadcasted_iota(jnp.int32, (1, 128), 1) % 8
    b_eff = (jnp.dot(b_sage_ref[...].reshape(1, -1), w_cls_ref[...],
                     preferred_element_type=jnp.float32)
             + b_cls_ref[...].reshape(1, -1))
    b_tile = (jnp.where(lane == 0, b_eff[0, 0], 0.0)
              + jnp.where(lane == 1, b_eff[0, 1], 0.0)
              + jnp.where(lane == 2, b_eff[0, 2], 0.0))
    cnt = jnp.where(lane == 0, _roll(a, -3),
                    jnp.where(lane == 1, _roll(a, -2), _roll(a, -1)))
    cnt = jnp.clip(cnt, 1.0, None)
    s = a / cnt + b_tile
    is_logit = lane < N_CLS
    sm = jnp.where(is_logit, s, -jnp.inf)
    m = jnp.maximum(
        jnp.maximum(jnp.maximum(_roll(sm, -2), _roll(sm, -1)), sm),
        jnp.maximum(_roll(sm, 1), _roll(sm, 2)))
    e = jnp.where(is_logit, jnp.exp(sm - m), 0.0)
    d = _roll(e, -2) + _roll(e, -1) + e + _roll(e, 1) + _roll(e, 2)
    out_ref[...] = jnp.where(is_logit, sm - m - jnp.log(d), 0.0)


def _finalize(acc_iv, w_cls, b_sage, b_cls):
    return pl.pallas_call(
        _finalize_body,
        out_shape=jax.ShapeDtypeStruct((N_PAD // 16, 128), jnp.float32),
    )(acc_iv, w_cls, b_sage, b_cls)


def kernel(x, n_id, edge_index, W_sage, b_sage, W_cls, b_cls):
    nid_pad = jnp.concatenate(
        [n_id, jnp.zeros((N_PAD - N_NODES,), jnp.int32)])
    zeros_init = jnp.zeros((N_PAD, 8), jnp.float32)

    xs = _gather_rows(x, nid_pad)
    z8 = _project(xs, W_sage, W_cls)
    acc = _edge_aggregate(edge_index, zeros_init, z8)
    res = _finalize(acc.reshape(NC, N_PAD // 16, 128), W_cls, b_sage, b_cls)
    return res.reshape(N_PAD, 8)[:N_NODES, :N_CLS]
